# Initial kernel scaffold; baseline (speedup 1.0000x reference)
import jax
import jax.numpy as jnp
from jax.experimental import pallas as pl


def kernel(x, edge_index, node_type, root_idx, params):
    raise NotImplementedError("write your pallas kernel here")


# speedup vs baseline: 40.7479x; 40.7479x over previous
import functools

import jax
import jax.numpy as jnp
from jax import lax
from jax.experimental import pallas as pl
from jax.experimental.pallas import tpu as pltpu
from jax.experimental.pallas import tpu_sc as plsc

N = 10000
NP = 10240
D = 128
NH = 4
E = 320000
SINK = N
NCORE = 2
NSUB = 16
NW = NCORE * NSUB
CH = 128
EPW = 10112
NCHUNK = EPW // CH
EPAD = NW * EPW
RPT = NP // NSUB
BLK = 2048


def _ln(h, g, b):
    m = jnp.mean(h, axis=-1, keepdims=True)
    v = jnp.mean((h - m) * (h - m), axis=-1, keepdims=True)
    return (h - m) / jnp.sqrt(v + 1e-5) * g + b


def _bheads(w4, rows):
    return jnp.concatenate(
        [jnp.broadcast_to(w4[:, h:h + 1], (rows, 32)) for h in range(NH)], axis=1)


def _att_logits(xl, af, df):
    parts = []
    for f in (af, df):
        for h in range(NH):
            sl = slice(h * 32, h * 32 + 32)
            parts.append(jnp.sum(xl[:, sl] * f[:, sl], axis=1, keepdims=True))
    parts.append(jnp.zeros((xl.shape[0], 8), jnp.float32))
    return jnp.concatenate(parts, axis=1)


def _enc_body(xb, ntb,
              iW1, ib1, ig1, ibt1, iW2, ib2, ig2, ibt2,
              oW1, ob1, og1, obt1, oW2, ob2, og2, obt2,
              cW, caf, cdf,
              out0_o, xl_o, att_o):
    x = xb[...]

    def enc(W1, b1, g1, bt1, W2, b2, g2, bt2):
        h1 = jnp.maximum(
            _ln(jnp.dot(x, W1[...], preferred_element_type=jnp.float32) + b1[...],
                g1[...], bt1[...]), 0.0)
        return jnp.maximum(
            _ln(jnp.dot(h1, W2[...], preferred_element_type=jnp.float32) + b2[...],
                g2[...], bt2[...]), 0.0)

    oi = enc(iW1, ib1, ig1, ibt1, iW2, ib2, ig2, ibt2)
    oo = enc(oW1, ob1, og1, obt1, oW2, ob2, og2, obt2)
    m = ntb[...]
    out0 = m * oi + (1.0 - m) * oo
    out0_o[...] = out0
    xl = jnp.dot(out0, cW[...], preferred_element_type=jnp.float32)
    xl_o[...] = xl
    att_o[...] = _att_logits(xl, caf[...], cdf[...])


def _mid_body(xl1b, attb, numa, numb, dena, denb,
              c1b_, c2W, c2af, c2df,
              oneh_o, xl2_o, att2_o):
    att = attb[...]
    al = att[:, 0:4] + att[:, 4:8]
    al = jnp.where(al > 0.0, al, 0.2 * al)
    ws = jnp.exp(al)
    den = dena[...][:, 0:4] + denb[...][:, 0:4] + ws + 1e-16
    xl1 = xl1b[...]
    num = numa[...] + numb[...] + xl1 * _bheads(ws, BLK)
    oneh = num / _bheads(den, BLK) + c1b_[...]
    oneh_o[...] = oneh
    xl2 = jnp.dot(oneh, c2W[...], preferred_element_type=jnp.float32)
    xl2_o[...] = xl2
    att2_o[...] = _att_logits(xl2, c2af[...], c2df[...])


def _root_body(root_ref, out0b, onehb, xl2b, att2b, numa, numb, dena, denb,
               c2b_, lwW, lwb, lng_, lnb_, gW, gb, gg, gbt,
               sW, sb, fW, fb, fg, fbt, ha,
               out_o):
    att = att2b[...].reshape(1, 16)
    al = att[:, 0:4] + att[:, 4:8]
    al = jnp.where(al > 0.0, al, 0.2 * al)
    ws = jnp.exp(al)
    den = dena[...].reshape(1, 16)[:, 0:4] + denb[...].reshape(1, 16)[:, 0:4] + ws + 1e-16
    xl2 = xl2b[...].reshape(1, D)
    num = numa[...].reshape(1, D) + numb[...].reshape(1, D) + xl2 * _bheads(ws, 1)
    two = num / _bheads(den, 1) + c2b_[...]
    out0 = out0b[...].reshape(1, D)
    oneh = onehb[...].reshape(1, D)
    hwl = jnp.dot(out0, lwW[...], preferred_element_type=jnp.float32) + lwb[...]
    hwl = hwl - jnp.max(hwl, axis=-1, keepdims=True)
    eh = jnp.exp(hwl)
    hw = eh / jnp.sum(eh, axis=-1, keepdims=True)
    out = hw[:, 0:1] * oneh + hw[:, 1:2] * two
    out = _ln(out, lng_[...], lnb_[...])
    gates = _ln(jnp.dot(out, gW[...], preferred_element_type=jnp.float32) + gb[...],
                gg[...], gbt[...])
    gates = 1.0 / (1.0 + jnp.exp(-gates))
    og = out * gates
    out = jnp.where(og > 0.0, og, jnp.exp(jnp.minimum(og, 0.0)) - 1.0)
    ident = jnp.dot(oneh, sW[...], preferred_element_type=jnp.float32) + sb[...]
    hav = ha[...]
    out = hav[:, 0:1] * out + hav[:, 1:2] * ident
    emb = _ln(jnp.dot(out, fW[...], preferred_element_type=jnp.float32) + fb[...],
              fg[...], fbt[...])
    nrm = jnp.maximum(jnp.sqrt(jnp.sum(emb * emb, axis=-1, keepdims=True)), 1e-12)
    out_o[...] = (emb / nrm / 0.1).reshape(1, 1, D)


def _edge_body(xl_hbm, att_hbm, src_hbm, dst_hbm, num_out, den_out,
               src_v, dst_v, rows_v, atts_v, attd_v, wf_v, wden_v,
               num_sp, den_sp, sem, sem2, sem3):
    cid = lax.axis_index("c")
    sid = lax.axis_index("s")
    wid = cid * NSUB + sid

    zero = jnp.zeros((16,), jnp.float32)

    def z1(i, _):
        rows_v[i >> 3, pl.ds((i & 7) * 16, 16)] = zero
        return 0

    lax.fori_loop(0, CH * 8, z1, 0)

    def z2(i, _):
        wden_v[i, pl.ds(0, 16)] = zero
        return 0

    lax.fori_loop(0, CH, z2, 0)

    base_r = sid * RPT
    for k in range(RPT // CH):
        pltpu.sync_copy(rows_v, num_sp.at[pl.ds(base_r + k * CH, CH)])
        pltpu.sync_copy(wden_v, den_sp.at[pl.ds(base_r + k * CH, CH)])
    plsc.subcore_barrier()

    lane = lax.iota(jnp.int32, 16)
    e_in_g = lane >> 2
    h_lane = lane & 3
    zi = jnp.zeros((16,), jnp.int32)
    ebase = wid * EPW

    def chunk_body(ci, _):
        off = ebase + ci * CH
        pltpu.sync_copy(src_hbm.at[pl.ds(off, CH)], src_v)
        pltpu.sync_copy(dst_hbm.at[pl.ds(off, CH)], dst_v)
        c1 = pltpu.async_copy(xl_hbm.at[src_v], rows_v, sem)
        c2 = pltpu.async_copy(att_hbm.at[src_v], atts_v, sem2)
        c3 = pltpu.async_copy(att_hbm.at[dst_v], attd_v, sem3)
        c2.wait()
        c3.wait()

        def grp(g, _):
            el = g * 4 + e_in_g
            a1 = plsc.load_gather(atts_v, [el, h_lane])
            a2 = plsc.load_gather(attd_v, [el, h_lane + 4])
            al = a1 + a2
            al = jnp.where(al > 0.0, al, 0.2 * al)
            wv = jnp.exp(al)
            wf_v[pl.ds(g * 16, 16)] = wv
            plsc.store_scatter(wden_v, [el, h_lane], wv)
            return 0

        lax.fori_loop(0, CH // 4, grp, 0)
        c1.wait()

        def edge(e, _):
            b4 = e * NH
            for h in range(NH):
                wb = plsc.load_gather(wf_v, [zi + (b4 + h)])
                for j in range(2):
                    col = h * 32 + j * 16
                    rows_v[e, pl.ds(col, 16)] = rows_v[e, pl.ds(col, 16)] * wb
            return 0

        lax.fori_loop(0, CH, edge, 0)

        pltpu.sync_copy(rows_v, num_sp.at[dst_v], add=True)
        pltpu.sync_copy(wden_v, den_sp.at[dst_v], add=True)
        return 0

    lax.fori_loop(0, NCHUNK, chunk_body, 0)

    plsc.subcore_barrier()
    for k in range(RPT // CH):
        r0 = base_r + k * CH
        pltpu.sync_copy(num_sp.at[pl.ds(r0, CH)], num_out.at[cid, pl.ds(r0, CH)])
        pltpu.sync_copy(den_sp.at[pl.ds(r0, CH)], den_out.at[cid, pl.ds(r0, CH)])


@functools.cache
def _edge_call_fn():
    mesh = plsc.VectorSubcoreMesh(
        core_axis_name="c", subcore_axis_name="s",
        num_cores=NCORE, num_subcores=NSUB)
    return pl.kernel(
        _edge_body,
        out_type=(
            jax.ShapeDtypeStruct((NCORE, NP, D), jnp.float32),
            jax.ShapeDtypeStruct((NCORE, NP, 16), jnp.float32),
        ),
        mesh=mesh,
        scratch_types=(
            pltpu.VMEM((CH,), jnp.int32),
            pltpu.VMEM((CH,), jnp.int32),
            pltpu.VMEM((CH, D), jnp.float32),
            pltpu.VMEM((CH, 16), jnp.float32),
            pltpu.VMEM((CH, 16), jnp.float32),
            pltpu.VMEM((CH * NH,), jnp.float32),
            pltpu.VMEM((CH, 16), jnp.float32),
            pltpu.VMEM_SHARED((NP, D), jnp.float32),
            pltpu.VMEM_SHARED((NP, 16), jnp.float32),
            pltpu.SemaphoreType.DMA,
            pltpu.SemaphoreType.DMA,
            pltpu.SemaphoreType.DMA,
        ),
        compiler_params=pltpu.CompilerParams(
            needs_layout_passes=False, use_tc_tiling_on_sc=False),
    )


def _edge_phase(xl, att, src, dst):
    return _edge_call_fn()(xl, att, src, dst)


def kernel(x, edge_index, node_type, root_idx, params):
    p = params
    f32 = jnp.float32
    xp = jnp.zeros((NP, D), f32).at[:N].set(x.astype(f32))
    nt = jnp.zeros((NP, 1), f32).at[:N, 0].set((node_type == 1).astype(f32))
    ntb = jnp.broadcast_to(nt, (NP, D))
    src = jnp.concatenate(
        [edge_index[0].astype(jnp.int32), jnp.full((EPAD - E,), SINK, jnp.int32)])
    dst = jnp.concatenate(
        [edge_index[1].astype(jnp.int32), jnp.full((EPAD - E,), SINK, jnp.int32)])

    def v(a):
        return a.reshape(1, -1)

    row_s = pl.BlockSpec((BLK, D), lambda i: (i, 0))
    den_s = pl.BlockSpec((BLK, 16), lambda i: (i, 0))
    w128 = pl.BlockSpec((D, D), lambda i: (0, 0))
    vec_s = pl.BlockSpec((1, D), lambda i: (0, 0))
    grid = (NP // BLK,)
    rowT = jax.ShapeDtypeStruct((NP, D), f32)
    attT = jax.ShapeDtypeStruct((NP, 16), f32)

    out0, xl1, att1 = pl.pallas_call(
        _enc_body,
        grid=grid,
        in_specs=[row_s, row_s,
                  w128, vec_s, vec_s, vec_s, w128, vec_s, vec_s, vec_s,
                  w128, vec_s, vec_s, vec_s, w128, vec_s, vec_s, vec_s,
                  w128, vec_s, vec_s],
        out_specs=[row_s, row_s, den_s],
        out_shape=[rowT, rowT, attT],
    )(xp, ntb,
      p['iW1'], v(p['ib1']), v(p['ig1']), v(p['ibt1']),
      p['iW2'], v(p['ib2']), v(p['ig2']), v(p['ibt2']),
      p['oW1'], v(p['ob1']), v(p['og1']), v(p['obt1']),
      p['oW2'], v(p['ob2']), v(p['og2']), v(p['obt2']),
      p['c1W'], v(p['c1as']), v(p['c1ad']))

    num1, den1 = _edge_phase(xl1, att1, src, dst)

    oneh, xl2, att2 = pl.pallas_call(
        _mid_body,
        grid=grid,
        in_specs=[row_s, den_s, row_s, row_s, den_s, den_s,
                  vec_s, w128, vec_s, vec_s],
        out_specs=[row_s, row_s, den_s],
        out_shape=[rowT, rowT, attT],
    )(xl1, att1, num1[0], num1[1], den1[0], den1[1],
      v(p['c1b']), p['c2W'], v(p['c2as']), v(p['c2ad']))

    num2, den2 = _edge_phase(xl2, att2, src, dst)

    R = root_idx.shape[0]
    g128 = pl.BlockSpec((1, 1, D), lambda i, r: (r[i], 0, 0))
    g16 = pl.BlockSpec((1, 1, 16), lambda i, r: (r[i], 0, 0))
    w128c = pl.BlockSpec((D, D), lambda i, r: (0, 0))
    vecc = pl.BlockSpec((1, D), lambda i, r: (0, 0))
    w2c = pl.BlockSpec((D, 2), lambda i, r: (0, 0))
    v2c = pl.BlockSpec((1, 2), lambda i, r: (0, 0))

    def r3(a):
        return a.reshape(a.shape[0], 1, a.shape[1])

    emb = pl.pallas_call(
        _root_body,
        grid_spec=pltpu.PrefetchScalarGridSpec(
            num_scalar_prefetch=1,
            grid=(R,),
            in_specs=[g128, g128, g128, g16, g128, g128, g16, g16,
                      vecc, w2c, v2c, vecc, vecc, w128c, vecc, vecc, vecc,
                      w128c, vecc, w128c, vecc, vecc, vecc, v2c],
            out_specs=pl.BlockSpec((1, 1, D), lambda i, r: (i, 0, 0)),
        ),
        out_shape=jax.ShapeDtypeStruct((R, 1, D), f32),
    )(root_idx.astype(jnp.int32),
      r3(out0), r3(oneh), r3(xl2), r3(att2),
      r3(num2[0]), r3(num2[1]), r3(den2[0]), r3(den2[1]),
      v(p['c2b']), p['lwW'], v(p['lwb']), v(p['lng']), v(p['lnb']),
      p['gW'], v(p['gb']), v(p['gg']), v(p['gbt']),
      p['sW'], v(p['sb']),
      p['fW'], v(p['fb']), v(p['fg']), v(p['fbt']), v(p['ha']))
    return emb.reshape(R, D)


# speedup vs baseline: 44.8067x; 1.0996x over previous
import functools

import jax
import jax.numpy as jnp
from jax import lax
from jax.experimental import pallas as pl
from jax.experimental.pallas import tpu as pltpu
from jax.experimental.pallas import tpu_sc as plsc

N = 10000
NP = 10112
D = 128
NH = 4
E = 320000
SINK = N
NCORE = 2
NSUB = 16
NW = NCORE * NSUB
CH = 128
EPW = 10240
NCHUNK = EPW // CH
EPAD = NW * EPW
RPT = NP // NSUB
BLK = 2048


def _ln(h, g, b):
    m = jnp.mean(h, axis=-1, keepdims=True)
    v = jnp.mean((h - m) * (h - m), axis=-1, keepdims=True)
    return (h - m) / jnp.sqrt(v + 1e-5) * g + b


def _bheads(w4, rows):
    return jnp.concatenate(
        [jnp.broadcast_to(w4[:, h:h + 1], (rows, 32)) for h in range(NH)], axis=1)


def _att_logits(xl, af, df):
    parts = []
    for f in (af, df):
        for h in range(NH):
            sl = slice(h * 32, h * 32 + 32)
            parts.append(jnp.sum(xl[:, sl] * f[:, sl], axis=1, keepdims=True))
    parts.append(jnp.zeros((xl.shape[0], 8), jnp.float32))
    return jnp.concatenate(parts, axis=1)


def _enc_body(xb, ntb,
              iW1, ib1, ig1, ibt1, iW2, ib2, ig2, ibt2,
              oW1, ob1, og1, obt1, oW2, ob2, og2, obt2,
              cW, caf, cdf,
              out0_o, xl_o, att_o):
    x = xb[...]

    def enc(W1, b1, g1, bt1, W2, b2, g2, bt2):
        h1 = jnp.maximum(
            _ln(jnp.dot(x, W1[...], preferred_element_type=jnp.float32) + b1[...],
                g1[...], bt1[...]), 0.0)
        return jnp.maximum(
            _ln(jnp.dot(h1, W2[...], preferred_element_type=jnp.float32) + b2[...],
                g2[...], bt2[...]), 0.0)

    oi = enc(iW1, ib1, ig1, ibt1, iW2, ib2, ig2, ibt2)
    oo = enc(oW1, ob1, og1, obt1, oW2, ob2, og2, obt2)
    m = ntb[...]
    out0 = m * oi + (1.0 - m) * oo
    out0_o[...] = out0
    xl = jnp.dot(out0, cW[...], preferred_element_type=jnp.float32)
    xl_o[...] = xl
    att_o[...] = _att_logits(xl, caf[...], cdf[...])


def _mid_body(xl1b, attb, numa, numb, dena, denb,
              c1b_, c2W, c2af, c2df,
              oneh_o, xl2_o, att2_o):
    att = attb[...]
    al = att[:, 0:4] + att[:, 4:8]
    al = jnp.where(al > 0.0, al, 0.2 * al)
    ws = jnp.exp(al)
    den = dena[...][:, 0:4] + denb[...][:, 0:4] + ws + 1e-16
    xl1 = xl1b[...]
    num = numa[...] + numb[...] + xl1 * _bheads(ws, BLK)
    oneh = num / _bheads(den, BLK) + c1b_[...]
    oneh_o[...] = oneh
    xl2 = jnp.dot(oneh, c2W[...], preferred_element_type=jnp.float32)
    xl2_o[...] = xl2
    att2_o[...] = _att_logits(xl2, c2af[...], c2df[...])


def _root_body(root_ref, out0b, onehb, xl2b, att2b, numa, numb, dena, denb,
               c2b_, lwW, lwb, lng_, lnb_, gW, gb, gg, gbt,
               sW, sb, fW, fb, fg, fbt, ha,
               out_o):
    att = att2b[...].reshape(1, 16)
    al = att[:, 0:4] + att[:, 4:8]
    al = jnp.where(al > 0.0, al, 0.2 * al)
    ws = jnp.exp(al)
    den = dena[...].reshape(1, 16)[:, 0:4] + denb[...].reshape(1, 16)[:, 0:4] + ws + 1e-16
    xl2 = xl2b[...].reshape(1, D)
    num = numa[...].reshape(1, D) + numb[...].reshape(1, D) + xl2 * _bheads(ws, 1)
    two = num / _bheads(den, 1) + c2b_[...]
    out0 = out0b[...].reshape(1, D)
    oneh = onehb[...].reshape(1, D)
    hwl = jnp.dot(out0, lwW[...], preferred_element_type=jnp.float32) + lwb[...]
    hwl = hwl - jnp.max(hwl, axis=-1, keepdims=True)
    eh = jnp.exp(hwl)
    hw = eh / jnp.sum(eh, axis=-1, keepdims=True)
    out = hw[:, 0:1] * oneh + hw[:, 1:2] * two
    out = _ln(out, lng_[...], lnb_[...])
    gates = _ln(jnp.dot(out, gW[...], preferred_element_type=jnp.float32) + gb[...],
                gg[...], gbt[...])
    gates = 1.0 / (1.0 + jnp.exp(-gates))
    og = out * gates
    out = jnp.where(og > 0.0, og, jnp.exp(jnp.minimum(og, 0.0)) - 1.0)
    ident = jnp.dot(oneh, sW[...], preferred_element_type=jnp.float32) + sb[...]
    hav = ha[...]
    out = hav[:, 0:1] * out + hav[:, 1:2] * ident
    emb = _ln(jnp.dot(out, fW[...], preferred_element_type=jnp.float32) + fb[...],
              fg[...], fbt[...])
    nrm = jnp.maximum(jnp.sqrt(jnp.sum(emb * emb, axis=-1, keepdims=True)), 1e-12)
    out_o[...] = (emb / nrm / 0.1).reshape(1, 1, D)


def _edge_body(xl_hbm, att_hbm, src_hbm, dst_hbm, num_out, den_out,
               src_v, dst_v, rows_v, atts_v, attd_v, wf_v, wden_v,
               num_sp, den_sp, sems):
    cid = lax.axis_index("c")
    sid = lax.axis_index("s")
    wid = cid * NSUB + sid

    zero = jnp.zeros((16,), jnp.float32)

    def z1(i, _):
        rows_v[0, i >> 3, pl.ds((i & 7) * 16, 16)] = zero
        return 0

    lax.fori_loop(0, CH * 8, z1, 0)

    def z2(i, _):
        wden_v[i, pl.ds(0, 16)] = zero
        return 0

    lax.fori_loop(0, CH, z2, 0)

    base_r = sid * RPT
    for k in range(RPT // CH):
        pltpu.sync_copy(rows_v.at[0], num_sp.at[pl.ds(base_r + k * CH, CH)])
        pltpu.sync_copy(wden_v, den_sp.at[pl.ds(base_r + k * CH, CH)])
    rem = RPT - (RPT // CH) * CH
    if rem:
        r0 = base_r + (RPT // CH) * CH
        pltpu.sync_copy(rows_v.at[0, pl.ds(0, rem)], num_sp.at[pl.ds(r0, rem)])
        pltpu.sync_copy(wden_v.at[pl.ds(0, rem % CH if rem <= CH else CH)],
                        den_sp.at[pl.ds(r0, rem)])
    plsc.subcore_barrier()

    lane = lax.iota(jnp.int32, 16)
    e_in_g = lane >> 2
    h_lane = lane & 3
    zi = jnp.zeros((16,), jnp.int32)
    ebase = wid * EPW

    def load_idx_and_fire(cidx, b):
        off = ebase + cidx * CH
        pltpu.sync_copy(src_hbm.at[pl.ds(off, CH)], src_v.at[b])
        pltpu.sync_copy(dst_hbm.at[pl.ds(off, CH)], dst_v.at[b])
        pltpu.async_copy(xl_hbm.at[src_v.at[b]], rows_v.at[b], sems.at[b])

    load_idx_and_fire(0, 0)
    load_idx_and_fire(1, 1)

    def process(cidx, b):
        pltpu.make_async_copy(
            xl_hbm.at[src_v.at[b]], rows_v.at[b], sems.at[b]).wait()
        pltpu.sync_copy(att_hbm.at[src_v.at[b]], atts_v)
        pltpu.sync_copy(att_hbm.at[dst_v.at[b]], attd_v)

        def grp(g, _):
            el = g * 4 + e_in_g
            a1 = plsc.load_gather(atts_v, [el, h_lane])
            a2 = plsc.load_gather(attd_v, [el, h_lane + 4])
            al = a1 + a2
            al = jnp.where(al > 0.0, al, 0.2 * al)
            wv = jnp.exp(al)
            wf_v[pl.ds(g * 16, 16)] = wv
            plsc.store_scatter(wden_v, [el, h_lane], wv)
            return 0

        lax.fori_loop(0, CH // 4, grp, 0)

        def edge(e, _):
            b4 = e * NH
            for h in range(NH):
                wb = plsc.load_gather(wf_v, [zi + (b4 + h)])
                for j in range(2):
                    col = h * 32 + j * 16
                    rows_v[b, e, pl.ds(col, 16)] = rows_v[b, e, pl.ds(col, 16)] * wb
            return 0

        lax.fori_loop(0, CH, edge, 0)

        pltpu.sync_copy(rows_v.at[b], num_sp.at[dst_v.at[b]], add=True)
        pltpu.sync_copy(wden_v, den_sp.at[dst_v.at[b]], add=True)
        load_idx_and_fire(jnp.minimum(cidx + 2, NCHUNK - 1), b)

    def chunk_pair(cp, _):
        process(cp * 2, 0)
        process(cp * 2 + 1, 1)
        return 0

    lax.fori_loop(0, NCHUNK // 2, chunk_pair, 0)

    for b in range(2):
        pltpu.make_async_copy(
            xl_hbm.at[src_v.at[b]], rows_v.at[b], sems.at[b]).wait()

    plsc.subcore_barrier()
    for k in range(RPT // CH):
        r0 = base_r + k * CH
        pltpu.sync_copy(num_sp.at[pl.ds(r0, CH)], num_out.at[cid, pl.ds(r0, CH)])
        pltpu.sync_copy(den_sp.at[pl.ds(r0, CH)], den_out.at[cid, pl.ds(r0, CH)])
    if rem:
        r0 = base_r + (RPT // CH) * CH
        pltpu.sync_copy(num_sp.at[pl.ds(r0, rem)], num_out.at[cid, pl.ds(r0, rem)])
        pltpu.sync_copy(den_sp.at[pl.ds(r0, rem)], den_out.at[cid, pl.ds(r0, rem)])


@functools.cache
def _edge_call_fn():
    mesh = plsc.VectorSubcoreMesh(
        core_axis_name="c", subcore_axis_name="s",
        num_cores=NCORE, num_subcores=NSUB)
    return pl.kernel(
        _edge_body,
        out_type=(
            jax.ShapeDtypeStruct((NCORE, NP, D), jnp.float32),
            jax.ShapeDtypeStruct((NCORE, NP, 16), jnp.float32),
        ),
        mesh=mesh,
        scratch_types=(
            pltpu.VMEM((2, CH), jnp.int32),
            pltpu.VMEM((2, CH), jnp.int32),
            pltpu.VMEM((2, CH, D), jnp.float32),
            pltpu.VMEM((CH, 16), jnp.float32),
            pltpu.VMEM((CH, 16), jnp.float32),
            pltpu.VMEM((CH * NH,), jnp.float32),
            pltpu.VMEM((CH, 16), jnp.float32),
            pltpu.VMEM_SHARED((NP, D), jnp.float32),
            pltpu.VMEM_SHARED((NP, 16), jnp.float32),
            pltpu.SemaphoreType.DMA((2,)),
        ),
        compiler_params=pltpu.CompilerParams(
            needs_layout_passes=False, use_tc_tiling_on_sc=False),
    )


def _edge_phase(xl, att, src, dst):
    return _edge_call_fn()(xl, att, src, dst)


def kernel(x, edge_index, node_type, root_idx, params):
    p = params
    f32 = jnp.float32
    xp = jnp.zeros((NP, D), f32).at[:N].set(x.astype(f32))
    nt = jnp.zeros((NP, 1), f32).at[:N, 0].set((node_type == 1).astype(f32))
    ntb = jnp.broadcast_to(nt, (NP, D))
    src = jnp.concatenate(
        [edge_index[0].astype(jnp.int32), jnp.full((EPAD - E,), SINK, jnp.int32)])
    dst = jnp.concatenate(
        [edge_index[1].astype(jnp.int32), jnp.full((EPAD - E,), SINK, jnp.int32)])

    def v(a):
        return a.reshape(1, -1)

    row_s = pl.BlockSpec((BLK, D), lambda i: (i, 0))
    den_s = pl.BlockSpec((BLK, 16), lambda i: (i, 0))
    w128 = pl.BlockSpec((D, D), lambda i: (0, 0))
    vec_s = pl.BlockSpec((1, D), lambda i: (0, 0))
    grid = ((NP + BLK - 1) // BLK,)
    rowT = jax.ShapeDtypeStruct((NP, D), f32)
    attT = jax.ShapeDtypeStruct((NP, 16), f32)

    out0, xl1, att1 = pl.pallas_call(
        _enc_body,
        grid=grid,
        in_specs=[row_s, row_s,
                  w128, vec_s, vec_s, vec_s, w128, vec_s, vec_s, vec_s,
                  w128, vec_s, vec_s, vec_s, w128, vec_s, vec_s, vec_s,
                  w128, vec_s, vec_s],
        out_specs=[row_s, row_s, den_s],
        out_shape=[rowT, rowT, attT],
    )(xp, ntb,
      p['iW1'], v(p['ib1']), v(p['ig1']), v(p['ibt1']),
      p['iW2'], v(p['ib2']), v(p['ig2']), v(p['ibt2']),
      p['oW1'], v(p['ob1']), v(p['og1']), v(p['obt1']),
      p['oW2'], v(p['ob2']), v(p['og2']), v(p['obt2']),
      p['c1W'], v(p['c1as']), v(p['c1ad']))

    num1, den1 = _edge_phase(xl1, att1, src, dst)

    oneh, xl2, att2 = pl.pallas_call(
        _mid_body,
        grid=grid,
        in_specs=[row_s, den_s, row_s, row_s, den_s, den_s,
                  vec_s, w128, vec_s, vec_s],
        out_specs=[row_s, row_s, den_s],
        out_shape=[rowT, rowT, attT],
    )(xl1, att1, num1[0], num1[1], den1[0], den1[1],
      v(p['c1b']), p['c2W'], v(p['c2as']), v(p['c2ad']))

    num2, den2 = _edge_phase(xl2, att2, src, dst)

    R = root_idx.shape[0]
    g128 = pl.BlockSpec((1, 1, D), lambda i, r: (r[i], 0, 0))
    g16 = pl.BlockSpec((1, 1, 16), lambda i, r: (r[i], 0, 0))
    w128c = pl.BlockSpec((D, D), lambda i, r: (0, 0))
    vecc = pl.BlockSpec((1, D), lambda i, r: (0, 0))
    w2c = pl.BlockSpec((D, 2), lambda i, r: (0, 0))
    v2c = pl.BlockSpec((1, 2), lambda i, r: (0, 0))

    def r3(a):
        return a.reshape(a.shape[0], 1, a.shape[1])

    emb = pl.pallas_call(
        _root_body,
        grid_spec=pltpu.PrefetchScalarGridSpec(
            num_scalar_prefetch=1,
            grid=(R,),
            in_specs=[g128, g128, g128, g16, g128, g128, g16, g16,
                      vecc, w2c, v2c, vecc, vecc, w128c, vecc, vecc, vecc,
                      w128c, vecc, w128c, vecc, vecc, vecc, v2c],
            out_specs=pl.BlockSpec((1, 1, D), lambda i, r: (i, 0, 0)),
        ),
        out_shape=jax.ShapeDtypeStruct((R, 1, D), f32),
    )(root_idx.astype(jnp.int32),
      r3(out0), r3(oneh), r3(xl2), r3(att2),
      r3(num2[0]), r3(num2[1]), r3(den2[0]), r3(den2[1]),
      v(p['c2b']), p['lwW'], v(p['lwb']), v(p['lng']), v(p['lnb']),
      p['gW'], v(p['gb']), v(p['gg']), v(p['gbt']),
      p['sW'], v(p['sb']),
      p['fW'], v(p['fb']), v(p['fg']), v(p['fbt']), v(p['ha']))
    return emb.reshape(R, D)


# speedup vs baseline: 59.2395x; 1.3221x over previous
import functools

import jax
import jax.numpy as jnp
from jax import lax
from jax.experimental import pallas as pl
from jax.experimental.pallas import tpu as pltpu
from jax.experimental.pallas import tpu_sc as plsc

N = 10000
NP = 10112
D = 128
NH = 4
E = 320000
SINK = N
NCORE = 2
NSUB = 16
NW = NCORE * NSUB
CH = 112
EPW = 10080
NCHUNK = EPW // CH
EPAD = NW * EPW
RPT = NP // NSUB
BLK = 2048


def _ln(h, g, b):
    m = jnp.mean(h, axis=-1, keepdims=True)
    v = jnp.mean((h - m) * (h - m), axis=-1, keepdims=True)
    return (h - m) / jnp.sqrt(v + 1e-5) * g + b


def _bheads(w4, rows):
    return jnp.concatenate(
        [jnp.broadcast_to(w4[:, h:h + 1], (rows, 32)) for h in range(NH)], axis=1)


def _att_logits(xl, af, df):
    parts = []
    for f in (af, df):
        for h in range(NH):
            sl = slice(h * 32, h * 32 + 32)
            parts.append(jnp.sum(xl[:, sl] * f[:, sl], axis=1, keepdims=True))
    parts.append(jnp.zeros((xl.shape[0], 8), jnp.float32))
    return jnp.concatenate(parts, axis=1)


def _enc_body(xb, ntb,
              iW1, ib1, ig1, ibt1, iW2, ib2, ig2, ibt2,
              oW1, ob1, og1, obt1, oW2, ob2, og2, obt2,
              cW, caf, cdf,
              out0_o, xl_o, att_o):
    x = xb[...]

    def enc(W1, b1, g1, bt1, W2, b2, g2, bt2):
        h1 = jnp.maximum(
            _ln(jnp.dot(x, W1[...], preferred_element_type=jnp.float32) + b1[...],
                g1[...], bt1[...]), 0.0)
        return jnp.maximum(
            _ln(jnp.dot(h1, W2[...], preferred_element_type=jnp.float32) + b2[...],
                g2[...], bt2[...]), 0.0)

    oi = enc(iW1, ib1, ig1, ibt1, iW2, ib2, ig2, ibt2)
    oo = enc(oW1, ob1, og1, obt1, oW2, ob2, og2, obt2)
    m = ntb[...]
    out0 = m * oi + (1.0 - m) * oo
    out0_o[...] = out0
    xl = jnp.dot(out0, cW[...], preferred_element_type=jnp.float32)
    xl_o[...] = xl
    att_o[...] = _att_logits(xl, caf[...], cdf[...])


def _mid_body(xl1b, attb, numa, numb, dena, denb,
              c1b_, c2W, c2af, c2df,
              oneh_o, xl2_o, att2_o):
    att = attb[...]
    al = att[:, 0:4] + att[:, 4:8]
    al = jnp.where(al > 0.0, al, 0.2 * al)
    ws = jnp.exp(al)
    den = dena[...][:, 0:4] + denb[...][:, 0:4] + ws + 1e-16
    xl1 = xl1b[...]
    num = numa[...] + numb[...] + xl1 * _bheads(ws, BLK)
    oneh = num / _bheads(den, BLK) + c1b_[...]
    oneh_o[...] = oneh
    xl2 = jnp.dot(oneh, c2W[...], preferred_element_type=jnp.float32)
    xl2_o[...] = xl2
    att2_o[...] = _att_logits(xl2, c2af[...], c2df[...])


def _root_body(root_ref, out0b, onehb, xl2b, att2b, numa, numb, dena, denb,
               c2b_, lwW, lwb, lng_, lnb_, gW, gb, gg, gbt,
               sW, sb, fW, fb, fg, fbt, ha,
               out_o):
    att = att2b[...].reshape(1, 16)
    al = att[:, 0:4] + att[:, 4:8]
    al = jnp.where(al > 0.0, al, 0.2 * al)
    ws = jnp.exp(al)
    den = dena[...].reshape(1, 16)[:, 0:4] + denb[...].reshape(1, 16)[:, 0:4] + ws + 1e-16
    xl2 = xl2b[...].reshape(1, D)
    num = numa[...].reshape(1, D) + numb[...].reshape(1, D) + xl2 * _bheads(ws, 1)
    two = num / _bheads(den, 1) + c2b_[...]
    out0 = out0b[...].reshape(1, D)
    oneh = onehb[...].reshape(1, D)
    hwl = jnp.dot(out0, lwW[...], preferred_element_type=jnp.float32) + lwb[...]
    hwl = hwl - jnp.max(hwl, axis=-1, keepdims=True)
    eh = jnp.exp(hwl)
    hw = eh / jnp.sum(eh, axis=-1, keepdims=True)
    out = hw[:, 0:1] * oneh + hw[:, 1:2] * two
    out = _ln(out, lng_[...], lnb_[...])
    gates = _ln(jnp.dot(out, gW[...], preferred_element_type=jnp.float32) + gb[...],
                gg[...], gbt[...])
    gates = 1.0 / (1.0 + jnp.exp(-gates))
    og = out * gates
    out = jnp.where(og > 0.0, og, jnp.exp(jnp.minimum(og, 0.0)) - 1.0)
    ident = jnp.dot(oneh, sW[...], preferred_element_type=jnp.float32) + sb[...]
    hav = ha[...]
    out = hav[:, 0:1] * out + hav[:, 1:2] * ident
    emb = _ln(jnp.dot(out, fW[...], preferred_element_type=jnp.float32) + fb[...],
              fg[...], fbt[...])
    nrm = jnp.maximum(jnp.sqrt(jnp.sum(emb * emb, axis=-1, keepdims=True)), 1e-12)
    out_o[...] = (emb / nrm / 0.1).reshape(1, 1, D)


def _edge_body(xl_hbm, att_hbm, src_hbm, dst_hbm, num_out, den_out,
               src_v, dst_v, rows_v, atts_v, attd_v, wf_v, wden_v,
               num_sp, den_sp, sems, sema, semb):
    cid = lax.axis_index("c")
    sid = lax.axis_index("s")
    wid = cid * NSUB + sid

    zero = jnp.zeros((16,), jnp.float32)

    def z1(i, _):
        rows_v[0, i >> 3, pl.ds((i & 7) * 16, 16)] = zero
        return 0

    lax.fori_loop(0, CH * 8, z1, 0)

    def z2(i, _):
        wden_v[i, pl.ds(0, 16)] = zero
        return 0

    lax.fori_loop(0, CH, z2, 0)

    base_r = sid * RPT
    for k in range(RPT // CH):
        pltpu.sync_copy(rows_v.at[0], num_sp.at[pl.ds(base_r + k * CH, CH)])
        pltpu.sync_copy(wden_v, den_sp.at[pl.ds(base_r + k * CH, CH)])
    rem = RPT - (RPT // CH) * CH
    if rem:
        r0 = base_r + (RPT // CH) * CH
        pltpu.sync_copy(rows_v.at[0, pl.ds(0, rem)], num_sp.at[pl.ds(r0, rem)])
        pltpu.sync_copy(wden_v.at[pl.ds(0, rem % CH if rem <= CH else CH)],
                        den_sp.at[pl.ds(r0, rem)])
    plsc.subcore_barrier()

    lane = lax.iota(jnp.int32, 16)
    e_in_g = lane >> 2
    h_lane = lane & 3
    zi = jnp.zeros((16,), jnp.int32)
    ebase = wid * EPW

    def load_idx_and_fire(cidx, b):
        off = ebase + cidx * CH
        pltpu.sync_copy(src_hbm.at[pl.ds(off, CH)], src_v.at[b])
        pltpu.sync_copy(dst_hbm.at[pl.ds(off, CH)], dst_v.at[b])
        pltpu.async_copy(xl_hbm.at[src_v.at[b]], rows_v.at[b], sems.at[b])
        pltpu.async_copy(att_hbm.at[src_v.at[b]], atts_v.at[b], sema.at[b])
        pltpu.async_copy(att_hbm.at[dst_v.at[b]], attd_v.at[b], semb.at[b])

    load_idx_and_fire(0, 0)
    load_idx_and_fire(1, 1)

    def process(cidx, b):
        pltpu.make_async_copy(
            xl_hbm.at[src_v.at[b]], rows_v.at[b], sems.at[b]).wait()
        pltpu.make_async_copy(
            att_hbm.at[src_v.at[b]], atts_v.at[b], sema.at[b]).wait()
        pltpu.make_async_copy(
            att_hbm.at[dst_v.at[b]], attd_v.at[b], semb.at[b]).wait()
        zb = zi + b

        def grp(g, _):
            el = g * 4 + e_in_g
            a1 = plsc.load_gather(atts_v, [zb, el, h_lane])
            a2 = plsc.load_gather(attd_v, [zb, el, h_lane + 4])
            al = a1 + a2
            al = jnp.where(al > 0.0, al, 0.2 * al)
            wv = jnp.exp(al)
            wf_v[pl.ds(g * 16, 16)] = wv
            plsc.store_scatter(wden_v, [el, h_lane], wv)
            return 0

        lax.fori_loop(0, CH // 4, grp, 0)

        def edge(e, _):
            b4 = e * NH
            for h in range(NH):
                wb = plsc.load_gather(wf_v, [zi + (b4 + h)])
                for j in range(2):
                    col = h * 32 + j * 16
                    rows_v[b, e, pl.ds(col, 16)] = rows_v[b, e, pl.ds(col, 16)] * wb
            return 0

        lax.fori_loop(0, CH, edge, 0)

        pltpu.sync_copy(rows_v.at[b], num_sp.at[dst_v.at[b]], add=True)
        pltpu.sync_copy(wden_v, den_sp.at[dst_v.at[b]], add=True)
        load_idx_and_fire(jnp.minimum(cidx + 2, NCHUNK - 1), b)

    def chunk_pair(cp, _):
        process(cp * 2, 0)
        process(cp * 2 + 1, 1)
        return 0

    lax.fori_loop(0, NCHUNK // 2, chunk_pair, 0)

    for b in range(2):
        pltpu.make_async_copy(
            xl_hbm.at[src_v.at[b]], rows_v.at[b], sems.at[b]).wait()
        pltpu.make_async_copy(
            att_hbm.at[src_v.at[b]], atts_v.at[b], sema.at[b]).wait()
        pltpu.make_async_copy(
            att_hbm.at[dst_v.at[b]], attd_v.at[b], semb.at[b]).wait()

    plsc.subcore_barrier()
    for k in range(RPT // CH):
        r0 = base_r + k * CH
        pltpu.sync_copy(num_sp.at[pl.ds(r0, CH)], num_out.at[cid, pl.ds(r0, CH)])
        pltpu.sync_copy(den_sp.at[pl.ds(r0, CH)], den_out.at[cid, pl.ds(r0, CH)])
    if rem:
        r0 = base_r + (RPT // CH) * CH
        pltpu.sync_copy(num_sp.at[pl.ds(r0, rem)], num_out.at[cid, pl.ds(r0, rem)])
        pltpu.sync_copy(den_sp.at[pl.ds(r0, rem)], den_out.at[cid, pl.ds(r0, rem)])


@functools.cache
def _edge_call_fn():
    mesh = plsc.VectorSubcoreMesh(
        core_axis_name="c", subcore_axis_name="s",
        num_cores=NCORE, num_subcores=NSUB)
    return pl.kernel(
        _edge_body,
        out_type=(
            jax.ShapeDtypeStruct((NCORE, NP, D), jnp.float32),
            jax.ShapeDtypeStruct((NCORE, NP, 16), jnp.float32),
        ),
        mesh=mesh,
        scratch_types=(
            pltpu.VMEM((2, CH), jnp.int32),
            pltpu.VMEM((2, CH), jnp.int32),
            pltpu.VMEM((2, CH, D), jnp.float32),
            pltpu.VMEM((2, CH, 16), jnp.float32),
            pltpu.VMEM((2, CH, 16), jnp.float32),
            pltpu.VMEM((CH * NH,), jnp.float32),
            pltpu.VMEM((CH, 16), jnp.float32),
            pltpu.VMEM_SHARED((NP, D), jnp.float32),
            pltpu.VMEM_SHARED((NP, 16), jnp.float32),
            pltpu.SemaphoreType.DMA((2,)),
            pltpu.SemaphoreType.DMA((2,)),
            pltpu.SemaphoreType.DMA((2,)),
        ),
        compiler_params=pltpu.CompilerParams(
            needs_layout_passes=False, use_tc_tiling_on_sc=False),
    )


def _edge_phase(xl, att, src, dst):
    return _edge_call_fn()(xl, att, src, dst)


def kernel(x, edge_index, node_type, root_idx, params):
    p = params
    f32 = jnp.float32
    xp = jnp.zeros((NP, D), f32).at[:N].set(x.astype(f32))
    nt = jnp.zeros((NP, 1), f32).at[:N, 0].set((node_type == 1).astype(f32))
    ntb = jnp.broadcast_to(nt, (NP, D))
    src = jnp.concatenate(
        [edge_index[0].astype(jnp.int32), jnp.full((EPAD - E,), SINK, jnp.int32)])
    dst = jnp.concatenate(
        [edge_index[1].astype(jnp.int32), jnp.full((EPAD - E,), SINK, jnp.int32)])

    def v(a):
        return a.reshape(1, -1)

    row_s = pl.BlockSpec((BLK, D), lambda i: (i, 0))
    den_s = pl.BlockSpec((BLK, 16), lambda i: (i, 0))
    w128 = pl.BlockSpec((D, D), lambda i: (0, 0))
    vec_s = pl.BlockSpec((1, D), lambda i: (0, 0))
    grid = ((NP + BLK - 1) // BLK,)
    rowT = jax.ShapeDtypeStruct((NP, D), f32)
    attT = jax.ShapeDtypeStruct((NP, 16), f32)

    out0, xl1, att1 = pl.pallas_call(
        _enc_body,
        grid=grid,
        in_specs=[row_s, row_s,
                  w128, vec_s, vec_s, vec_s, w128, vec_s, vec_s, vec_s,
                  w128, vec_s, vec_s, vec_s, w128, vec_s, vec_s, vec_s,
                  w128, vec_s, vec_s],
        out_specs=[row_s, row_s, den_s],
        out_shape=[rowT, rowT, attT],
    )(xp, ntb,
      p['iW1'], v(p['ib1']), v(p['ig1']), v(p['ibt1']),
      p['iW2'], v(p['ib2']), v(p['ig2']), v(p['ibt2']),
      p['oW1'], v(p['ob1']), v(p['og1']), v(p['obt1']),
      p['oW2'], v(p['ob2']), v(p['og2']), v(p['obt2']),
      p['c1W'], v(p['c1as']), v(p['c1ad']))

    num1, den1 = _edge_phase(xl1, att1, src, dst)

    oneh, xl2, att2 = pl.pallas_call(
        _mid_body,
        grid=grid,
        in_specs=[row_s, den_s, row_s, row_s, den_s, den_s,
                  vec_s, w128, vec_s, vec_s],
        out_specs=[row_s, row_s, den_s],
        out_shape=[rowT, rowT, attT],
    )(xl1, att1, num1[0], num1[1], den1[0], den1[1],
      v(p['c1b']), p['c2W'], v(p['c2as']), v(p['c2ad']))

    num2, den2 = _edge_phase(xl2, att2, src, dst)

    R = root_idx.shape[0]
    g128 = pl.BlockSpec((1, 1, D), lambda i, r: (r[i], 0, 0))
    g16 = pl.BlockSpec((1, 1, 16), lambda i, r: (r[i], 0, 0))
    w128c = pl.BlockSpec((D, D), lambda i, r: (0, 0))
    vecc = pl.BlockSpec((1, D), lambda i, r: (0, 0))
    w2c = pl.BlockSpec((D, 2), lambda i, r: (0, 0))
    v2c = pl.BlockSpec((1, 2), lambda i, r: (0, 0))

    def r3(a):
        return a.reshape(a.shape[0], 1, a.shape[1])

    emb = pl.pallas_call(
        _root_body,
        grid_spec=pltpu.PrefetchScalarGridSpec(
            num_scalar_prefetch=1,
            grid=(R,),
            in_specs=[g128, g128, g128, g16, g128, g128, g16, g16,
                      vecc, w2c, v2c, vecc, vecc, w128c, vecc, vecc, vecc,
                      w128c, vecc, w128c, vecc, vecc, vecc, v2c],
            out_specs=pl.BlockSpec((1, 1, D), lambda i, r: (i, 0, 0)),
        ),
        out_shape=jax.ShapeDtypeStruct((R, 1, D), f32),
    )(root_idx.astype(jnp.int32),
      r3(out0), r3(oneh), r3(xl2), r3(att2),
      r3(num2[0]), r3(num2[1]), r3(den2[0]), r3(den2[1]),
      v(p['c2b']), p['lwW'], v(p['lwb']), v(p['lng']), v(p['lnb']),
      p['gW'], v(p['gb']), v(p['gg']), v(p['gbt']),
      p['sW'], v(p['sb']),
      p['fW'], v(p['fb']), v(p['fg']), v(p['fbt']), v(p['ha']))
    return emb.reshape(R, D)


# speedup vs baseline: 74.3415x; 1.2549x over previous
import functools

import jax
import jax.numpy as jnp
from jax import lax
from jax.experimental import pallas as pl
from jax.experimental.pallas import tpu as pltpu
from jax.experimental.pallas import tpu_sc as plsc

N = 10000
NP = 10112
D = 128
NH = 4
E = 320000
SINK = N
NCORE = 2
NSUB = 16
NW = NCORE * NSUB
CH = 112
EPW = 10080
NCHUNK = EPW // CH
EPAD = NW * EPW
RPT = NP // NSUB
BLK = 2048


def _ln(h, g, b):
    m = jnp.mean(h, axis=-1, keepdims=True)
    v = jnp.mean((h - m) * (h - m), axis=-1, keepdims=True)
    return (h - m) / jnp.sqrt(v + 1e-5) * g + b


def _bheads(w4, rows):
    return jnp.concatenate(
        [jnp.broadcast_to(w4[:, h:h + 1], (rows, 32)) for h in range(NH)], axis=1)


def _att_logits(xl, af, df):
    parts = []
    for f in (af, df):
        for h in range(NH):
            sl = slice(h * 32, h * 32 + 32)
            parts.append(jnp.sum(xl[:, sl] * f[:, sl], axis=1, keepdims=True))
    parts.append(jnp.zeros((xl.shape[0], 8), jnp.float32))
    return jnp.concatenate(parts, axis=1)


def _enc_body(xb, ntb,
              iW1, ib1, ig1, ibt1, iW2, ib2, ig2, ibt2,
              oW1, ob1, og1, obt1, oW2, ob2, og2, obt2,
              cW, caf, cdf,
              out0_o, xl_o, att_o):
    x = xb[...]

    def enc(W1, b1, g1, bt1, W2, b2, g2, bt2):
        h1 = jnp.maximum(
            _ln(jnp.dot(x, W1[...], preferred_element_type=jnp.float32) + b1[...],
                g1[...], bt1[...]), 0.0)
        return jnp.maximum(
            _ln(jnp.dot(h1, W2[...], preferred_element_type=jnp.float32) + b2[...],
                g2[...], bt2[...]), 0.0)

    oi = enc(iW1, ib1, ig1, ibt1, iW2, ib2, ig2, ibt2)
    oo = enc(oW1, ob1, og1, obt1, oW2, ob2, og2, obt2)
    m = ntb[...]
    out0 = m * oi + (1.0 - m) * oo
    out0_o[...] = out0
    xl = jnp.dot(out0, cW[...], preferred_element_type=jnp.float32)
    xl_o[...] = xl
    att_o[...] = _att_logits(xl, caf[...], cdf[...])


def _mid_body(xl1b, attb, numa, numb, dena, denb,
              c1b_, c2W, c2af, c2df,
              oneh_o, xl2_o, att2_o):
    att = attb[...]
    al = att[:, 0:4] + att[:, 4:8]
    al = jnp.where(al > 0.0, al, 0.2 * al)
    ws = jnp.exp(al)
    den = dena[...][:, 0:4] + denb[...][:, 0:4] + ws + 1e-16
    xl1 = xl1b[...]
    num = numa[...] + numb[...] + xl1 * _bheads(ws, BLK)
    oneh = num / _bheads(den, BLK) + c1b_[...]
    oneh_o[...] = oneh
    xl2 = jnp.dot(oneh, c2W[...], preferred_element_type=jnp.float32)
    xl2_o[...] = xl2
    att2_o[...] = _att_logits(xl2, c2af[...], c2df[...])


def _root_body(root_ref, out0b, onehb, xl2b, att2b, numa, numb, dena, denb,
               c2b_, lwW, lwb, lng_, lnb_, gW, gb, gg, gbt,
               sW, sb, fW, fb, fg, fbt, ha,
               out_o):
    att = att2b[...].reshape(1, 16)
    al = att[:, 0:4] + att[:, 4:8]
    al = jnp.where(al > 0.0, al, 0.2 * al)
    ws = jnp.exp(al)
    den = dena[...].reshape(1, 16)[:, 0:4] + denb[...].reshape(1, 16)[:, 0:4] + ws + 1e-16
    xl2 = xl2b[...].reshape(1, D)
    num = numa[...].reshape(1, D) + numb[...].reshape(1, D) + xl2 * _bheads(ws, 1)
    two = num / _bheads(den, 1) + c2b_[...]
    out0 = out0b[...].reshape(1, D)
    oneh = onehb[...].reshape(1, D)
    hwl = jnp.dot(out0, lwW[...], preferred_element_type=jnp.float32) + lwb[...]
    hwl = hwl - jnp.max(hwl, axis=-1, keepdims=True)
    eh = jnp.exp(hwl)
    hw = eh / jnp.sum(eh, axis=-1, keepdims=True)
    out = hw[:, 0:1] * oneh + hw[:, 1:2] * two
    out = _ln(out, lng_[...], lnb_[...])
    gates = _ln(jnp.dot(out, gW[...], preferred_element_type=jnp.float32) + gb[...],
                gg[...], gbt[...])
    gates = 1.0 / (1.0 + jnp.exp(-gates))
    og = out * gates
    out = jnp.where(og > 0.0, og, jnp.exp(jnp.minimum(og, 0.0)) - 1.0)
    ident = jnp.dot(oneh, sW[...], preferred_element_type=jnp.float32) + sb[...]
    hav = ha[...]
    out = hav[:, 0:1] * out + hav[:, 1:2] * ident
    emb = _ln(jnp.dot(out, fW[...], preferred_element_type=jnp.float32) + fb[...],
              fg[...], fbt[...])
    nrm = jnp.maximum(jnp.sqrt(jnp.sum(emb * emb, axis=-1, keepdims=True)), 1e-12)
    out_o[...] = (emb / nrm / 0.1).reshape(1, 1, D)


def _edge_body(xl_hbm, att_hbm, src_hbm, dst_hbm, num_out, den_out,
               src_v, dst_v, rows_v, atts_v, attd_v, wf_v, wden_v,
               num_sp, den_sp, sems, sema, semb):
    cid = lax.axis_index("c")
    sid = lax.axis_index("s")
    wid = cid * NSUB + sid

    zero = jnp.zeros((16,), jnp.float32)

    def z1(i, _):
        rows_v[0, i >> 3, pl.ds((i & 7) * 16, 16)] = zero
        return 0

    lax.fori_loop(0, CH * 8, z1, 0)

    def z2(i, _):
        wden_v[i, pl.ds(0, 16)] = zero
        return 0

    lax.fori_loop(0, CH, z2, 0)

    base_r = sid * RPT
    for k in range(RPT // CH):
        pltpu.sync_copy(rows_v.at[0], num_sp.at[pl.ds(base_r + k * CH, CH)])
        pltpu.sync_copy(wden_v, den_sp.at[pl.ds(base_r + k * CH, CH)])
    rem = RPT - (RPT // CH) * CH
    if rem:
        r0 = base_r + (RPT // CH) * CH
        pltpu.sync_copy(rows_v.at[0, pl.ds(0, rem)], num_sp.at[pl.ds(r0, rem)])
        pltpu.sync_copy(wden_v.at[pl.ds(0, rem % CH if rem <= CH else CH)],
                        den_sp.at[pl.ds(r0, rem)])
    plsc.subcore_barrier()

    lane = lax.iota(jnp.int32, 16)
    e_in_g = lane >> 2
    h_lane = lane & 3
    zi = jnp.zeros((16,), jnp.int32)
    ebase = wid * EPW

    def load_idx_and_fire(cidx, b):
        off = ebase + cidx * CH
        pltpu.sync_copy(src_hbm.at[pl.ds(off, CH)], src_v.at[b])
        pltpu.sync_copy(dst_hbm.at[pl.ds(off, CH)], dst_v.at[b])
        pltpu.async_copy(xl_hbm.at[src_v.at[b]], rows_v.at[b], sems.at[b])
        pltpu.async_copy(att_hbm.at[src_v.at[b]], atts_v.at[b], sema.at[b])
        pltpu.async_copy(att_hbm.at[dst_v.at[b]], attd_v.at[b], semb.at[b])

    load_idx_and_fire(0, 0)
    load_idx_and_fire(1, 1)

    def process(cidx, b):
        pltpu.make_async_copy(
            xl_hbm.at[src_v.at[b]], rows_v.at[b], sems.at[b]).wait()
        pltpu.make_async_copy(
            att_hbm.at[src_v.at[b]], atts_v.at[b], sema.at[b]).wait()
        pltpu.make_async_copy(
            att_hbm.at[dst_v.at[b]], attd_v.at[b], semb.at[b]).wait()
        zb = zi + b

        def grp(g, _):
            el = g * 4 + e_in_g
            a1 = plsc.load_gather(atts_v, [zb, el, h_lane])
            a2 = plsc.load_gather(attd_v, [zb, el, h_lane + 4])
            al = a1 + a2
            al = jnp.where(al > 0.0, al, 0.2 * al)
            wv = jnp.exp(al)
            wf_v[pl.ds(g * 16, 16)] = wv
            plsc.store_scatter(wden_v, [el, h_lane], wv)
            return 0

        lax.fori_loop(0, CH // 4, grp, 0)

        @plsc.parallel_loop(0, CH, unroll=4)
        def _(e):
            b4 = e * NH
            for h in range(NH):
                wb = plsc.load_gather(wf_v, [zi + (b4 + h)])
                for j in range(2):
                    col = h * 32 + j * 16
                    rows_v[b, e, pl.ds(col, 16)] = rows_v[b, e, pl.ds(col, 16)] * wb

        pltpu.sync_copy(rows_v.at[b], num_sp.at[dst_v.at[b]], add=True)
        pltpu.sync_copy(wden_v, den_sp.at[dst_v.at[b]], add=True)
        load_idx_and_fire(jnp.minimum(cidx + 2, NCHUNK - 1), b)

    def chunk_pair(cp, _):
        process(cp * 2, 0)
        process(cp * 2 + 1, 1)
        return 0

    lax.fori_loop(0, NCHUNK // 2, chunk_pair, 0)

    for b in range(2):
        pltpu.make_async_copy(
            xl_hbm.at[src_v.at[b]], rows_v.at[b], sems.at[b]).wait()
        pltpu.make_async_copy(
            att_hbm.at[src_v.at[b]], atts_v.at[b], sema.at[b]).wait()
        pltpu.make_async_copy(
            att_hbm.at[dst_v.at[b]], attd_v.at[b], semb.at[b]).wait()

    plsc.subcore_barrier()
    for k in range(RPT // CH):
        r0 = base_r + k * CH
        pltpu.sync_copy(num_sp.at[pl.ds(r0, CH)], num_out.at[cid, pl.ds(r0, CH)])
        pltpu.sync_copy(den_sp.at[pl.ds(r0, CH)], den_out.at[cid, pl.ds(r0, CH)])
    if rem:
        r0 = base_r + (RPT // CH) * CH
        pltpu.sync_copy(num_sp.at[pl.ds(r0, rem)], num_out.at[cid, pl.ds(r0, rem)])
        pltpu.sync_copy(den_sp.at[pl.ds(r0, rem)], den_out.at[cid, pl.ds(r0, rem)])


@functools.cache
def _edge_call_fn():
    mesh = plsc.VectorSubcoreMesh(
        core_axis_name="c", subcore_axis_name="s",
        num_cores=NCORE, num_subcores=NSUB)
    return pl.kernel(
        _edge_body,
        out_type=(
            jax.ShapeDtypeStruct((NCORE, NP, D), jnp.float32),
            jax.ShapeDtypeStruct((NCORE, NP, 16), jnp.float32),
        ),
        mesh=mesh,
        scratch_types=(
            pltpu.VMEM((2, CH), jnp.int32),
            pltpu.VMEM((2, CH), jnp.int32),
            pltpu.VMEM((2, CH, D), jnp.float32),
            pltpu.VMEM((2, CH, 16), jnp.float32),
            pltpu.VMEM((2, CH, 16), jnp.float32),
            pltpu.VMEM((CH * NH,), jnp.float32),
            pltpu.VMEM((CH, 16), jnp.float32),
            pltpu.VMEM_SHARED((NP, D), jnp.float32),
            pltpu.VMEM_SHARED((NP, 16), jnp.float32),
            pltpu.SemaphoreType.DMA((2,)),
            pltpu.SemaphoreType.DMA((2,)),
            pltpu.SemaphoreType.DMA((2,)),
        ),
        compiler_params=pltpu.CompilerParams(
            needs_layout_passes=False, use_tc_tiling_on_sc=False),
    )


def _edge_phase(xl, att, src, dst):
    return _edge_call_fn()(xl, att, src, dst)


def kernel(x, edge_index, node_type, root_idx, params):
    p = params
    f32 = jnp.float32
    xp = jnp.zeros((NP, D), f32).at[:N].set(x.astype(f32))
    nt = jnp.zeros((NP, 1), f32).at[:N, 0].set((node_type == 1).astype(f32))
    ntb = jnp.broadcast_to(nt, (NP, D))
    src = jnp.concatenate(
        [edge_index[0].astype(jnp.int32), jnp.full((EPAD - E,), SINK, jnp.int32)])
    dst = jnp.concatenate(
        [edge_index[1].astype(jnp.int32), jnp.full((EPAD - E,), SINK, jnp.int32)])

    def v(a):
        return a.reshape(1, -1)

    row_s = pl.BlockSpec((BLK, D), lambda i: (i, 0))
    den_s = pl.BlockSpec((BLK, 16), lambda i: (i, 0))
    w128 = pl.BlockSpec((D, D), lambda i: (0, 0))
    vec_s = pl.BlockSpec((1, D), lambda i: (0, 0))
    grid = ((NP + BLK - 1) // BLK,)
    rowT = jax.ShapeDtypeStruct((NP, D), f32)
    attT = jax.ShapeDtypeStruct((NP, 16), f32)

    out0, xl1, att1 = pl.pallas_call(
        _enc_body,
        grid=grid,
        in_specs=[row_s, row_s,
                  w128, vec_s, vec_s, vec_s, w128, vec_s, vec_s, vec_s,
                  w128, vec_s, vec_s, vec_s, w128, vec_s, vec_s, vec_s,
                  w128, vec_s, vec_s],
        out_specs=[row_s, row_s, den_s],
        out_shape=[rowT, rowT, attT],
    )(xp, ntb,
      p['iW1'], v(p['ib1']), v(p['ig1']), v(p['ibt1']),
      p['iW2'], v(p['ib2']), v(p['ig2']), v(p['ibt2']),
      p['oW1'], v(p['ob1']), v(p['og1']), v(p['obt1']),
      p['oW2'], v(p['ob2']), v(p['og2']), v(p['obt2']),
      p['c1W'], v(p['c1as']), v(p['c1ad']))

    num1, den1 = _edge_phase(xl1, att1, src, dst)

    oneh, xl2, att2 = pl.pallas_call(
        _mid_body,
        grid=grid,
        in_specs=[row_s, den_s, row_s, row_s, den_s, den_s,
                  vec_s, w128, vec_s, vec_s],
        out_specs=[row_s, row_s, den_s],
        out_shape=[rowT, rowT, attT],
    )(xl1, att1, num1[0], num1[1], den1[0], den1[1],
      v(p['c1b']), p['c2W'], v(p['c2as']), v(p['c2ad']))

    num2, den2 = _edge_phase(xl2, att2, src, dst)

    R = root_idx.shape[0]
    g128 = pl.BlockSpec((1, 1, D), lambda i, r: (r[i], 0, 0))
    g16 = pl.BlockSpec((1, 1, 16), lambda i, r: (r[i], 0, 0))
    w128c = pl.BlockSpec((D, D), lambda i, r: (0, 0))
    vecc = pl.BlockSpec((1, D), lambda i, r: (0, 0))
    w2c = pl.BlockSpec((D, 2), lambda i, r: (0, 0))
    v2c = pl.BlockSpec((1, 2), lambda i, r: (0, 0))

    def r3(a):
        return a.reshape(a.shape[0], 1, a.shape[1])

    emb = pl.pallas_call(
        _root_body,
        grid_spec=pltpu.PrefetchScalarGridSpec(
            num_scalar_prefetch=1,
            grid=(R,),
            in_specs=[g128, g128, g128, g16, g128, g128, g16, g16,
                      vecc, w2c, v2c, vecc, vecc, w128c, vecc, vecc, vecc,
                      w128c, vecc, w128c, vecc, vecc, vecc, v2c],
            out_specs=pl.BlockSpec((1, 1, D), lambda i, r: (i, 0, 0)),
        ),
        out_shape=jax.ShapeDtypeStruct((R, 1, D), f32),
    )(root_idx.astype(jnp.int32),
      r3(out0), r3(oneh), r3(xl2), r3(att2),
      r3(num2[0]), r3(num2[1]), r3(den2[0]), r3(den2[1]),
      v(p['c2b']), p['lwW'], v(p['lwb']), v(p['lng']), v(p['lnb']),
      p['gW'], v(p['gb']), v(p['gg']), v(p['gbt']),
      p['sW'], v(p['sb']),
      p['fW'], v(p['fb']), v(p['fg']), v(p['fbt']), v(p['ha']))
    return emb.reshape(R, D)
